# baseline (speedup 1.0000x reference)
import functools

import jax
import jax.numpy as jnp
from jax import lax
from jax.experimental import pallas as pl
from jax.experimental.pallas import tpu as pltpu
from jax.experimental.pallas import tpu_sc as plsc

_ALPHA = 1.0
_BETA = 0.2
_LANES = 1024
_SEGS = 100000
_N = 6400000

_NW = 32
_SEG_PER = 3128
_SEG_LAST = _SEGS - (_NW - 1) * _SEG_PER
_TROWS = 3136
_CH = 2048


def _fold_unfold_body(x_ref, out_ref, hist_ref):
    x = x_ref[:]
    bm, lanes = x.shape
    lane = jax.lax.broadcasted_iota(jnp.int32, x.shape, 1) % 4
    is_c = lane == 3
    inf = jnp.float32(jnp.inf)

    def one_iter(x):
        l1 = jnp.roll(x, -1, axis=1)
        l2 = jnp.roll(x, -2, axis=1)
        fv = jnp.minimum(x, jnp.minimum(l1, l2))
        x1 = jnp.where(is_c, x + _ALPHA * jnp.roll(fv, 3, axis=1), x)
        cb = jnp.where(
            lane == 0,
            jnp.roll(x1, -3, axis=1),
            jnp.where(lane == 1, jnp.roll(x1, -2, axis=1), jnp.roll(x1, -1, axis=1)),
        )
        x2 = jnp.where(is_c, x1, x1 + _BETA * cb)
        fmask = jnp.where(lane == 0, fv, inf)
        fold = jnp.min(fmask.reshape(bm, lanes // 4, 4), axis=2)
        return x2, fold

    x2, fold0 = one_iter(x)
    x4, fold1 = one_iter(x2)
    out_ref[:] = x4
    hist_ref[:] = jnp.stack([fold0, fold1])


def _run_fold_unfold(states):
    n = states.shape[0]
    flat = states.reshape(n * 4 // _LANES, _LANES)
    m = flat.shape[0]
    bm = 200
    grid = m // bm
    out_flat, hist = pl.pallas_call(
        _fold_unfold_body,
        grid=(grid,),
        in_specs=[pl.BlockSpec((bm, _LANES), lambda i: (i, 0))],
        out_specs=[
            pl.BlockSpec((bm, _LANES), lambda i: (i, 0)),
            pl.BlockSpec((2, bm, _LANES // 4), lambda i: (0, i, 0)),
        ],
        out_shape=[
            jax.ShapeDtypeStruct((m, _LANES), jnp.float32),
            jax.ShapeDtypeStruct((2, m, _LANES // 4), jnp.float32),
        ],
        compiler_params=pltpu.CompilerParams(
            dimension_semantics=("parallel",),
        ),
    )(flat)
    updated = out_flat.reshape(n, 4)
    fold_history = hist.reshape(2, n)
    return updated, fold_history


def _segmax_body(upd_hbm, ids_hbm, start_hbm, agg_hbm, vbuf, idbuf, table, sbuf):
    wid = lax.axis_index("s") * 2 + lax.axis_index("c")
    seg0 = wid * _SEG_PER
    nseg = jnp.where(wid == _NW - 1, _SEG_LAST, _SEG_PER)
    neg_inf = jnp.full((16,), -jnp.inf, dtype=jnp.float32)
    iota = lax.iota(jnp.int32, 16)

    def init_step(i, _):
        table[pl.ds(i * 16, 16)] = neg_inf
        return 0

    lax.fori_loop(0, (_TROWS * 4) // 16, init_step, 0)

    pltpu.sync_copy(start_hbm.at[pl.ds(seg0, 8)], sbuf.at[pl.ds(0, 8)])
    pltpu.sync_copy(start_hbm.at[pl.ds(seg0 + nseg, 8)], sbuf.at[pl.ds(8, 8)])
    sv = sbuf[pl.ds(0, 16)]
    row_lo = lax.reduce_max(jnp.where(iota == 0, sv, 0), axes=(0,))
    row_hi = lax.reduce_max(jnp.where(iota == 8, sv, 0), axes=(0,))

    abase0 = (row_lo // 8) * 8
    nch = (row_hi - abase0 + _CH - 1) // _CH

    def chunk_step(k, _):
        abase = abase0 + k * _CH
        abase_c = jnp.minimum(abase, _N - _CH)
        lo_eff = jnp.maximum(row_lo, abase)
        pltpu.sync_copy(upd_hbm.at[pl.ds(abase_c * 4, _CH * 4)], vbuf)
        pltpu.sync_copy(ids_hbm.at[pl.ds(abase_c, _CH)], idbuf)

        def group_step(g, _):
            p0 = g * 16
            pos = p0 + iota
            ids = idbuf[pl.ds(p0, 16)]
            gidx = abase_c + pos
            mask = (gidx >= lo_eff) & (gidx < row_hi)
            lid = jnp.clip(ids - seg0, 0, _TROWS - 1)
            base4 = pos * 4
            tbase = lid * 4
            vals = [plsc.load_gather(vbuf, [base4 + c]) for c in range(4)]
            def lane_step(j, _):
                mj = mask & (iota == j)
                for c in range(4):
                    t = plsc.load_gather(table, [tbase + c])
                    plsc.store_scatter(
                        table, [tbase + c], jnp.maximum(t, vals[c]), mask=mj
                    )
                return 0

            lax.fori_loop(0, 16, lane_step, 0)
            return 0

        lax.fori_loop(0, _CH // 16, group_step, 0)
        return 0

    lax.fori_loop(0, nch, chunk_step, 0)

    @pl.when(wid < _NW - 1)
    def _():
        pltpu.sync_copy(
            table.at[pl.ds(0, _SEG_PER * 4)],
            agg_hbm.at[pl.ds(seg0 * 4, _SEG_PER * 4)],
        )

    @pl.when(wid == _NW - 1)
    def _():
        pltpu.sync_copy(
            table.at[pl.ds(0, _SEG_LAST * 4)],
            agg_hbm.at[pl.ds(seg0 * 4, _SEG_LAST * 4)],
        )


def _run_segmax(updated_flat, batch, start_padded):
    mesh = plsc.VectorSubcoreMesh(core_axis_name="c", subcore_axis_name="s")
    k = functools.partial(
        pl.kernel,
        mesh=mesh,
        out_type=jax.ShapeDtypeStruct((_SEGS * 4,), jnp.float32),
        scratch_types=[
            pltpu.VMEM((_CH * 4,), jnp.float32),
            pltpu.VMEM((_CH,), jnp.int32),
            pltpu.VMEM((_TROWS * 4,), jnp.float32),
            pltpu.VMEM((16,), jnp.int32),
        ],
        compiler_params=pltpu.CompilerParams(needs_layout_passes=False),
    )(_segmax_body)
    return k(updated_flat, batch, start_padded)


def kernel(states, batch, iterations):
    updated, fold_history = _run_fold_unfold(states)
    start = jnp.searchsorted(
        batch, jnp.arange(_SEGS + 1, dtype=jnp.int32), side="left"
    ).astype(jnp.int32)
    start_padded = jnp.pad(start, (0, 7), constant_values=_N)
    agg_flat = _run_segmax(updated.reshape(-1), batch, start_padded)
    aggregated = agg_flat.reshape(_SEGS, 4)
    center_out = aggregated[..., 3]
    loops_out = aggregated[..., :3]
    return (updated, aggregated, center_out, loops_out, fold_history)


# speedup vs baseline: 1.0104x; 1.0104x over previous
import functools

import jax
import jax.numpy as jnp
from jax import lax
from jax.experimental import pallas as pl
from jax.experimental.pallas import tpu as pltpu
from jax.experimental.pallas import tpu_sc as plsc

_ALPHA = 1.0
_BETA = 0.2
_LANES = 1024
_SEGS = 100000
_N = 6400000

_NW = 32
_SEG_PER = 3128
_SEG_LAST = _SEGS - (_NW - 1) * _SEG_PER
_TROWS = 3136
_CH = 2048


def _fold_unfold_body(x_ref, out_ref, hist_ref):
    x = x_ref[:]
    bm, lanes = x.shape
    lane = jax.lax.broadcasted_iota(jnp.int32, x.shape, 1) % 4
    is_c = lane == 3
    inf = jnp.float32(jnp.inf)

    def one_iter(x):
        l1 = jnp.roll(x, -1, axis=1)
        l2 = jnp.roll(x, -2, axis=1)
        fv = jnp.minimum(x, jnp.minimum(l1, l2))
        x1 = jnp.where(is_c, x + _ALPHA * jnp.roll(fv, 3, axis=1), x)
        cb = jnp.where(
            lane == 0,
            jnp.roll(x1, -3, axis=1),
            jnp.where(lane == 1, jnp.roll(x1, -2, axis=1), jnp.roll(x1, -1, axis=1)),
        )
        x2 = jnp.where(is_c, x1, x1 + _BETA * cb)
        fmask = jnp.where(lane == 0, fv, inf)
        fold = jnp.min(fmask.reshape(bm, lanes // 4, 4), axis=2)
        return x2, fold

    x2, fold0 = one_iter(x)
    x4, fold1 = one_iter(x2)
    out_ref[:] = x4
    hist_ref[:] = jnp.stack([fold0, fold1])


def _run_fold_unfold(states):
    n = states.shape[0]
    flat = states.reshape(n * 4 // _LANES, _LANES)
    m = flat.shape[0]
    bm = 200
    grid = m // bm
    out_flat, hist = pl.pallas_call(
        _fold_unfold_body,
        grid=(grid,),
        in_specs=[pl.BlockSpec((bm, _LANES), lambda i: (i, 0))],
        out_specs=[
            pl.BlockSpec((bm, _LANES), lambda i: (i, 0)),
            pl.BlockSpec((2, bm, _LANES // 4), lambda i: (0, i, 0)),
        ],
        out_shape=[
            jax.ShapeDtypeStruct((m, _LANES), jnp.float32),
            jax.ShapeDtypeStruct((2, m, _LANES // 4), jnp.float32),
        ],
        compiler_params=pltpu.CompilerParams(
            dimension_semantics=("parallel",),
        ),
    )(flat)
    updated = out_flat.reshape(n, 4)
    fold_history = hist.reshape(2, n)
    return updated, fold_history


def _segmax_body(upd_hbm, ids_hbm, start_hbm, agg_hbm, vbuf, idbuf, table, sbuf):
    wid = lax.axis_index("s") * 2 + lax.axis_index("c")
    seg0 = wid * _SEG_PER
    nseg = jnp.where(wid == _NW - 1, _SEG_LAST, _SEG_PER)
    neg_inf = jnp.full((16,), -jnp.inf, dtype=jnp.float32)
    iota = lax.iota(jnp.int32, 16)

    def init_step(i, _):
        table[pl.ds(i * 16, 16)] = neg_inf
        return 0

    lax.fori_loop(0, (_TROWS * 4) // 16, init_step, 0)

    pltpu.sync_copy(start_hbm.at[pl.ds(seg0, 8)], sbuf.at[pl.ds(0, 8)])
    pltpu.sync_copy(start_hbm.at[pl.ds(seg0 + nseg, 8)], sbuf.at[pl.ds(8, 8)])
    sv = sbuf[pl.ds(0, 16)]
    row_lo = lax.reduce_max(jnp.where(iota == 0, sv, 0), axes=(0,))
    row_hi = lax.reduce_max(jnp.where(iota == 8, sv, 0), axes=(0,))

    abase0 = (row_lo // 8) * 8
    nch = (row_hi - abase0 + _CH - 1) // _CH

    neg = jnp.float32(-jnp.inf)
    big = jnp.int32(1 << 30)

    def flush(cur, accs):
        idx0 = jnp.clip(cur, 0, _TROWS - 1) * 4
        fmask = (iota == 0) & (cur >= 0)
        for c in range(4):
            mx = lax.reduce_max(accs[c], axes=(0,))
            idxv = jnp.full((16,), idx0 + c, dtype=jnp.int32)
            t = plsc.load_gather(table, [idxv])
            plsc.store_scatter(
                table, [idxv], jnp.maximum(t, jnp.full((16,), mx)), mask=fmask
            )

    def chunk_step(k, carry):
        abase = abase0 + k * _CH
        abase_c = jnp.minimum(abase, _N - _CH)
        lo_eff = jnp.maximum(row_lo, abase)
        pltpu.sync_copy(upd_hbm.at[pl.ds(abase_c * 4, _CH * 4)], vbuf)
        pltpu.sync_copy(ids_hbm.at[pl.ds(abase_c, _CH)], idbuf)

        def group_step(g, carry):
            cur, a0, a1, a2, a3 = carry
            accs = [a0, a1, a2, a3]
            p0 = g * 16
            pos = p0 + iota
            ids = idbuf[pl.ds(p0, 16)]
            gidx = abase_c + pos
            mask = (gidx >= lo_eff) & (gidx < row_hi)
            lid = ids - seg0
            base4 = pos * 4
            vals = [plsc.load_gather(vbuf, [base4 + c]) for c in range(4)]
            first = lax.reduce_min(jnp.where(mask, lid, big), axes=(0,))
            last = lax.reduce_max(jnp.where(mask, lid, -1), axes=(0,))
            hit = jnp.all(mask) & (first == last) & (first == cur)

            def fast_path(cur, accs):
                return (cur, *[jnp.maximum(a, v) for a, v in zip(accs, vals)])

            def slow_path(cur, accs):
                flush(cur, accs)
                def wbody(rem):
                    r = lax.reduce_min(jnp.where(rem, lid, big), axes=(0,))
                    rm = rem & (lid == r)
                    ridx0 = jnp.clip(r, 0, _TROWS - 1) * 4
                    smask = (iota == 0) & (r < big)
                    for c in range(4):
                        mx = lax.reduce_max(
                            jnp.where(rm, vals[c], neg), axes=(0,)
                        )
                        idxv = jnp.full((16,), ridx0 + c, dtype=jnp.int32)
                        t = plsc.load_gather(table, [idxv])
                        plsc.store_scatter(
                            table,
                            [idxv],
                            jnp.maximum(t, jnp.full((16,), mx)),
                            mask=smask,
                        )
                    return rem & jnp.logical_not(rm)

                rem0 = mask & (lid != last)
                lax.while_loop(lambda rem: jnp.any(rem), wbody, rem0)
                lm = mask & (lid == last)
                return (last, *[jnp.where(lm, v, neg) for v in vals])

            return lax.cond(hit, fast_path, slow_path, cur, accs)

        return lax.fori_loop(0, _CH // 16, group_step, carry)

    neg16 = jnp.full((16,), -jnp.inf, dtype=jnp.float32)
    carry0 = (jnp.int32(-1), neg16, neg16, neg16, neg16)
    cur, a0, a1, a2, a3 = lax.fori_loop(0, nch, chunk_step, carry0)
    flush(cur, [a0, a1, a2, a3])

    @pl.when(wid < _NW - 1)
    def _():
        pltpu.sync_copy(
            table.at[pl.ds(0, _SEG_PER * 4)],
            agg_hbm.at[pl.ds(seg0 * 4, _SEG_PER * 4)],
        )

    @pl.when(wid == _NW - 1)
    def _():
        pltpu.sync_copy(
            table.at[pl.ds(0, _SEG_LAST * 4)],
            agg_hbm.at[pl.ds(seg0 * 4, _SEG_LAST * 4)],
        )


def _run_segmax(updated_flat, batch, start_padded):
    mesh = plsc.VectorSubcoreMesh(core_axis_name="c", subcore_axis_name="s")
    k = functools.partial(
        pl.kernel,
        mesh=mesh,
        out_type=jax.ShapeDtypeStruct((_SEGS * 4,), jnp.float32),
        scratch_types=[
            pltpu.VMEM((_CH * 4,), jnp.float32),
            pltpu.VMEM((_CH,), jnp.int32),
            pltpu.VMEM((_TROWS * 4,), jnp.float32),
            pltpu.VMEM((16,), jnp.int32),
        ],
        compiler_params=pltpu.CompilerParams(needs_layout_passes=False),
    )(_segmax_body)
    return k(updated_flat, batch, start_padded)


def kernel(states, batch, iterations):
    updated, fold_history = _run_fold_unfold(states)
    start = jnp.searchsorted(
        batch, jnp.arange(_SEGS + 1, dtype=jnp.int32), side="left"
    ).astype(jnp.int32)
    start_padded = jnp.pad(start, (0, 7), constant_values=_N)
    agg_flat = _run_segmax(updated.reshape(-1), batch, start_padded)
    aggregated = agg_flat.reshape(_SEGS, 4)
    center_out = aggregated[..., 3]
    loops_out = aggregated[..., :3]
    return (updated, aggregated, center_out, loops_out, fold_history)
